# Initial kernel scaffold; baseline (speedup 1.0000x reference)
import jax
import jax.numpy as jnp
from jax.experimental import pallas as pl


def kernel(ctx_xyz, ctx_tokens, pred_tokens, fp_W, fp_b, ec_W1, ec_b1, ec_W2, ec_b2, co_W1, co_b1, co_W2, co_b2, po_W1, po_b1, po_W2, po_b2, fo_W1, fo_b1, fo_W2, fo_b2):
    raise NotImplementedError("write your pallas kernel here")


# speedup vs baseline: 6.3235x; 6.3235x over previous
import jax
import jax.numpy as jnp
from jax import lax
from jax.experimental import pallas as pl
from jax.experimental.pallas import tpu as pltpu
from jax.experimental.pallas import tpu_sc as plsc

_B, _P, _M = 2, 512, 4
_C = 128
_UP_R = 12
_PRED_R = 16
_CTX_RADIUS = 0.02
_PRED_RADIUS = 0.05
_K = 8
_NF = 256
_N_CTX = _P * _UP_R
_N_ALL = _N_CTX + _M * _PRED_R
_N_ALLP = 6272

_bf = jnp.bfloat16


def _mm(a, b):
    return jnp.dot(a.astype(_bf), b.astype(_bf),
                   preferred_element_type=jnp.float32)


def _proj_body(tok_ref, w_ref, b_ref, out_ref):
    out_ref[...] = _mm(tok_ref[...], w_ref[...]) + b_ref[...]


def _feat_proj(ctx_tokens, fp_W, fp_b):
    d_in = ctx_tokens.shape[-1]
    return pl.pallas_call(
        _proj_body,
        grid=(_B,),
        in_specs=[
            pl.BlockSpec((None, _P, d_in), lambda b: (b, 0, 0)),
            pl.BlockSpec((d_in, _C), lambda b: (0, 0)),
            pl.BlockSpec((1, _C), lambda b: (0, 0)),
        ],
        out_specs=pl.BlockSpec((None, _P, _C), lambda b: (b, 0, 0)),
        out_shape=jax.ShapeDtypeStruct((_B, _P, _C), jnp.float32),
    )(ctx_tokens, fp_W, fp_b.reshape(1, _C))


def _knn_body(xyz_ref, xyzt_ref, idx_ref):
    tr = xyz_ref.shape[0]
    npts = xyzt_ref.shape[1]
    r = pl.program_id(1)
    xr = xyz_ref[...]
    xt = xyzt_ref[...]
    sr = jnp.sum(xr * xr, axis=1, keepdims=True)
    sc = jnp.sum(xt * xt, axis=0, keepdims=True)
    xr16 = xr.astype(_bf).astype(jnp.float32)
    xt16 = xt.astype(_bf).astype(jnp.float32)
    cross = (xr16[:, 0:1] * xt16[0:1, :]
             + xr16[:, 1:2] * xt16[1:2, :]
             + xr16[:, 2:3] * xt16[2:3, :])
    d = (sr + sc) - 2.0 * cross
    colio = lax.broadcasted_iota(jnp.int32, (tr, npts), 1)
    rowg = r * tr + lax.broadcasted_iota(jnp.int32, (tr, 1), 0)
    d = jnp.where(colio == rowg, 1e9, d)
    cols = []
    for _ in range(_K):
        mn = jnp.min(d, axis=1, keepdims=True)
        cand = jnp.where(d == mn, colio, npts)
        j = jnp.min(cand, axis=1, keepdims=True)
        cols.append(j)
        d = jnp.where(colio == j, 1e9, d)
    idx_ref[...] = jnp.concatenate(cols, axis=1)


def _knn(xyz, xyzt, npts):
    tr = 128
    return pl.pallas_call(
        _knn_body,
        grid=(_B, npts // tr),
        in_specs=[
            pl.BlockSpec((None, tr, 3), lambda b, r: (b, r, 0)),
            pl.BlockSpec((None, 3, npts), lambda b, r: (b, 0, 0)),
        ],
        out_specs=pl.BlockSpec((None, tr, _K), lambda b, r: (b, r, 0)),
        out_shape=jax.ShapeDtypeStruct((_B, npts, _K), jnp.int32),
    )(xyz, xyzt)


def _gather_rows(table, idx_flat):
    e_total = idx_flat.shape[0]
    width = table.shape[1]
    nw = 32
    ew = e_total // nw
    ch = 128 if ew % 128 == 0 else 112
    nch = ew // ch
    mesh = plsc.VectorSubcoreMesh(core_axis_name="c", subcore_axis_name="s")

    def body(table_hbm, idx_hbm, out_hbm, idx_v, rows_v, sem):
        wid = lax.axis_index("s") * 2 + lax.axis_index("c")
        base = pl.multiple_of(wid * ew, 8)
        pltpu.sync_copy(idx_hbm.at[pl.ds(base, ew)], idx_v)

        def step(ci, carry):
            off = pl.multiple_of(ci * ch, 8)
            pltpu.async_copy(
                table_hbm.at[idx_v.at[pl.ds(off, ch)]], rows_v, sem
            ).wait()
            pltpu.sync_copy(rows_v, out_hbm.at[pl.ds(base + off, ch)])
            return carry

        lax.fori_loop(0, nch, step, 0)

    gk = pl.kernel(
        body,
        out_type=jax.ShapeDtypeStruct((e_total, width), jnp.float32),
        mesh=mesh,
        scratch_types=[
            pltpu.VMEM((ew,), jnp.int32),
            pltpu.VMEM((ch, width), jnp.float32),
            pltpu.SemaphoreType.DMA,
        ],
    )
    return gk(table, idx_flat)


def _edge_body(feat_ref, xyz_ref, g_ref, w1tf_ref, w1tx_ref, w1bf_ref,
               w1bx_ref, b1_ref, w2_ref, b2_ref, cw1_ref, cb1_ref, cw2_ref,
               cb2_ref, feato_ref, xyzo_ref):
    feat = feat_ref[...]
    xyz = xyz_ref[...]
    t_xi = (_mm(feat, w1tf_ref[...]) + _mm(xyz, w1tx_ref[...])
            + b1_ref[...])
    w1bf = w1bf_ref[...]
    w1bx = w1bx_ref[...]
    w2 = w2_ref[...]
    m = jnp.full((feat.shape[0], _C), -1e30, jnp.float32)
    for k in range(_K):
        gk = g_ref[k]
        df = gk[:, :_C] - feat
        dx = gk[:, _C:_C + 3] - xyz
        h = jnp.maximum(t_xi + _mm(df, w1bf) + _mm(dx, w1bx), 0.0)
        s = _mm(h, w2)
        m = jnp.maximum(m, s)
    feat_o = m + b2_ref[...]
    feato_ref[...] = feat_o
    y = jnp.maximum(_mm(feat_o, cw1_ref[...]) + cb1_ref[...], 0.0)
    xyzo_ref[...] = xyz + _mm(y, cw2_ref[...]) + cb2_ref[...]


def _edge_mlp(feat, xyz, g, w1tf, w1tx, w1bf, w1bx, b1, w2, b2,
              cw1, cb1, cw2, cb2, npts, tre):
    return pl.pallas_call(
        _edge_body,
        grid=(_B, npts // tre),
        in_specs=[
            pl.BlockSpec((None, tre, _C), lambda b, t: (b, t, 0)),
            pl.BlockSpec((None, tre, 3), lambda b, t: (b, t, 0)),
            pl.BlockSpec((None, _K, tre, _NF), lambda b, t: (b, 0, t, 0)),
            pl.BlockSpec((_C, _C), lambda b, t: (0, 0)),
            pl.BlockSpec((3, _C), lambda b, t: (0, 0)),
            pl.BlockSpec((_C, _C), lambda b, t: (0, 0)),
            pl.BlockSpec((3, _C), lambda b, t: (0, 0)),
            pl.BlockSpec((1, _C), lambda b, t: (0, 0)),
            pl.BlockSpec((_C, _C), lambda b, t: (0, 0)),
            pl.BlockSpec((1, _C), lambda b, t: (0, 0)),
            pl.BlockSpec((_C, _C), lambda b, t: (0, 0)),
            pl.BlockSpec((1, _C), lambda b, t: (0, 0)),
            pl.BlockSpec((_C, 3), lambda b, t: (0, 0)),
            pl.BlockSpec((1, 3), lambda b, t: (0, 0)),
        ],
        out_specs=[
            pl.BlockSpec((None, tre, _C), lambda b, t: (b, t, 0)),
            pl.BlockSpec((None, tre, 3), lambda b, t: (b, t, 0)),
        ],
        out_shape=[
            jax.ShapeDtypeStruct((_B, npts, _C), jnp.float32),
            jax.ShapeDtypeStruct((_B, npts, 3), jnp.float32),
        ],
    )(feat, xyz, g, w1tf, w1tx, w1bf, w1bx, b1.reshape(1, _C), w2,
      b2.reshape(1, _C), cw1, cb1.reshape(1, _C), cw2, cb2.reshape(1, 3))


def _edgeconv_big(xyz, feat, npts, w1tf, w1tx, w1bf, w1bx, b1, w2, b2,
                  cw1, cb1, cw2, cb2):
    tre = 512 if npts % 512 == 0 else 448
    xyzt = jnp.transpose(xyz, (0, 2, 1))
    idx = _knn(xyz, xyzt, npts)
    offs = (jnp.arange(_B, dtype=jnp.int32) * npts)[:, None, None]
    idx_flat = jnp.transpose(idx + offs, (0, 2, 1)).reshape(_B * _K * npts)
    nfp = jnp.concatenate(
        [feat, xyz, jnp.zeros((_B, npts, _NF - _C - 3), jnp.float32)],
        axis=-1)
    g = _gather_rows(nfp.reshape(_B * npts, _NF), idx_flat)
    g = g.reshape(_B, _K, npts, _NF)
    return _edge_mlp(feat, xyz, g, w1tf, w1tx, w1bf, w1bx, b1, w2, b2,
                     cw1, cb1, cw2, cb2, npts, tre)


def _pred_body(xyz_ref, xyzt_ref, feat_ref, w1tf_ref, w1tx_ref, w1bx_ref,
               b1_ref, w2_ref, b2_ref, pw1_ref, pb1_ref, pw2_ref,
               pb2_ref, feato_ref, xyzo_ref):
    x = xyz_ref[...]
    xt = xyzt_ref[...]
    f = feat_ref[...]
    t1 = (_mm(f, w1tf_ref[...]) + _mm(x, w1tx_ref[...])
          + b1_ref[...])
    sqr = jnp.sum(x * x, axis=1, keepdims=True)
    sqc = jnp.sum(xt * xt, axis=0, keepdims=True)
    d = (sqr + sqc) - 2.0 * _mm(x, xt)
    colio = lax.broadcasted_iota(jnp.int32, (_PRED_R, _PRED_R), 1)
    rowio = lax.broadcasted_iota(jnp.int32, (_PRED_R, _PRED_R), 0)
    d = jnp.where(colio == rowio, 1e9, d)
    sel_mask = colio < 0
    for _ in range(_K):
        mn = jnp.min(d, axis=1, keepdims=True)
        cand = jnp.where(d == mn, colio, _PRED_R)
        j = jnp.min(cand, axis=1, keepdims=True)
        sel = colio == j
        sel_mask = jnp.logical_or(sel_mask, sel)
        d = jnp.where(sel, 1e9, d)
    w1bx = w1bx_ref[...]
    w2 = w2_ref[...]
    out = jnp.full((_PRED_R, _C), -1e30, jnp.float32)
    for j in range(_PRED_R):
        dx = x[j:j + 1, :] - x
        h = jnp.maximum(t1 + _mm(dx, w1bx), 0.0)
        s = _mm(h, w2)
        ok = sel_mask[:, j:j + 1]
        out = jnp.where(ok, jnp.maximum(out, s), out)
    feat_o = out + b2_ref[...]
    feato_ref[...] = feat_o
    y = jnp.maximum(_mm(feat_o, pw1_ref[...]) + pb1_ref[...], 0.0)
    xyzo_ref[...] = x + _mm(y, pw2_ref[...]) + pb2_ref[...]


def _pred_blocks(xyz_p, feat_p, w1tf, w1tx, w1bx, b1, w2, b2,
                 pw1, pb1, pw2, pb2):
    g = _B * _M
    xyz_g = xyz_p.reshape(g, _PRED_R, 3)
    xyzt_g = jnp.transpose(xyz_g, (0, 2, 1))
    feat_g = feat_p.reshape(g, 1, _C)
    wspec = lambda shp: pl.BlockSpec(shp, lambda i: tuple(0 for _ in shp))
    feat_o, xyz_o = pl.pallas_call(
        _pred_body,
        grid=(g,),
        in_specs=[
            pl.BlockSpec((None, _PRED_R, 3), lambda i: (i, 0, 0)),
            pl.BlockSpec((None, 3, _PRED_R), lambda i: (i, 0, 0)),
            pl.BlockSpec((None, 1, _C), lambda i: (i, 0, 0)),
            wspec((_C, _C)), wspec((3, _C)), wspec((3, _C)),
            wspec((1, _C)), wspec((_C, _C)), wspec((1, _C)),
            wspec((_C, _C)), wspec((1, _C)), wspec((_C, 3)), wspec((1, 3)),
        ],
        out_specs=[
            pl.BlockSpec((None, _PRED_R, _C), lambda i: (i, 0, 0)),
            pl.BlockSpec((None, _PRED_R, 3), lambda i: (i, 0, 0)),
        ],
        out_shape=[
            jax.ShapeDtypeStruct((g, _PRED_R, _C), jnp.float32),
            jax.ShapeDtypeStruct((g, _PRED_R, 3), jnp.float32),
        ],
    )(xyz_g, xyzt_g, feat_g, w1tf, w1tx, w1bx, b1.reshape(1, _C), w2,
      b2.reshape(1, _C), pw1, pb1.reshape(1, _C), pw2, pb2.reshape(1, 3))
    return feat_o.reshape(_B, _M, _PRED_R, _C), xyz_o.reshape(_B, _M, _PRED_R, 3)


def kernel(ctx_xyz, ctx_tokens, pred_tokens, fp_W, fp_b, ec_W1, ec_b1, ec_W2,
           ec_b2, co_W1, co_b1, co_W2, co_b2, po_W1, po_b1, po_W2, po_b2,
           fo_W1, fo_b1, fo_W2, fo_b2):
    nkey = jax.random.key(42)
    noise = jax.random.normal(jax.random.fold_in(nkey, 0),
                              (_B, _P, _UP_R, 3), dtype=jnp.float32)
    noise = noise / (jnp.linalg.norm(noise, axis=-1, keepdims=True) + 1e-6)
    noise = noise * _CTX_RADIUS
    xyz_ctx = (ctx_xyz[:, :, None, :] + noise).reshape(_B, _N_CTX, 3)

    ctx_feat = _feat_proj(ctx_tokens, fp_W, fp_b)
    feat_ctx = jnp.broadcast_to(
        ctx_feat[:, :, None, :], (_B, _P, _UP_R, _C)
    ).reshape(_B, _N_CTX, _C)

    in_dim = _C + 3
    w1tf = ec_W1[:_C]
    w1tx = ec_W1[_C:in_dim]
    w1bf = ec_W1[in_dim:in_dim + _C]
    w1bx = ec_W1[in_dim + _C:]

    feat1, xyz1 = _edgeconv_big(xyz_ctx, feat_ctx, _N_CTX, w1tf, w1tx, w1bf,
                                w1bx, ec_b1, ec_W2, ec_b2, co_W1, co_b1,
                                co_W2, co_b2)

    anchor = jnp.mean(ctx_xyz, axis=1)
    pns = []
    for m in range(_M):
        pn = jax.random.normal(jax.random.fold_in(nkey, 100 + m),
                               (_B, _PRED_R, 3), dtype=jnp.float32)
        pn = pn / (jnp.linalg.norm(pn, axis=-1, keepdims=True) + 1e-6)
        pns.append(pn * _PRED_RADIUS)
    xyz_p = anchor[:, None, None, :] + jnp.stack(pns, axis=1)
    feat_p, xyz_p_new = _pred_blocks(xyz_p, pred_tokens, w1tf, w1tx, w1bx,
                                     ec_b1, ec_W2, ec_b2, po_W1, po_b1,
                                     po_W2, po_b2)

    pad = _N_ALLP - _N_ALL
    sent = (1e4 + 10.0 * jnp.arange(pad, dtype=jnp.float32))
    sent = jnp.broadcast_to(sent[None, :, None], (_B, pad, 3))
    xyz_all = jnp.concatenate(
        [xyz1, xyz_p_new.reshape(_B, _M * _PRED_R, 3), sent], axis=1)
    feat_all = jnp.concatenate(
        [feat1, feat_p.reshape(_B, _M * _PRED_R, _C),
         jnp.zeros((_B, pad, _C), jnp.float32)], axis=1)

    _, xyz2 = _edgeconv_big(xyz_all, feat_all, _N_ALLP, w1tf, w1tx, w1bf,
                            w1bx, ec_b1, ec_W2, ec_b2, fo_W1, fo_b1, fo_W2,
                            fo_b2)
    return xyz2[:, :_N_ALL]


# speedup vs baseline: 6.3950x; 1.0113x over previous
import jax
import jax.numpy as jnp
from jax import lax
from jax.experimental import pallas as pl
from jax.experimental.pallas import tpu as pltpu
from jax.experimental.pallas import tpu_sc as plsc

_B, _P, _M = 2, 512, 4
_C = 128
_UP_R = 12
_PRED_R = 16
_CTX_RADIUS = 0.02
_PRED_RADIUS = 0.05
_K = 8
_N_CTX = _P * _UP_R
_N_ALL = _N_CTX + _M * _PRED_R
_N_ALLP = 6272

_bf = jnp.bfloat16


def _mm(a, b):
    return jnp.dot(a.astype(_bf), b.astype(_bf),
                   preferred_element_type=jnp.float32)


def _proj_body(tok_ref, w_ref, b_ref, out_ref):
    out_ref[...] = _mm(tok_ref[...], w_ref[...]) + b_ref[...]


def _feat_proj(ctx_tokens, fp_W, fp_b):
    d_in = ctx_tokens.shape[-1]
    return pl.pallas_call(
        _proj_body,
        grid=(_B,),
        in_specs=[
            pl.BlockSpec((None, _P, d_in), lambda b: (b, 0, 0)),
            pl.BlockSpec((d_in, _C), lambda b: (0, 0)),
            pl.BlockSpec((1, _C), lambda b: (0, 0)),
        ],
        out_specs=pl.BlockSpec((None, _P, _C), lambda b: (b, 0, 0)),
        out_shape=jax.ShapeDtypeStruct((_B, _P, _C), jnp.float32),
    )(ctx_tokens, fp_W, fp_b.reshape(1, _C))


def _knn_body(xyz_ref, xyzt_ref, idx_ref):
    tr = xyz_ref.shape[0]
    npts = xyzt_ref.shape[1]
    r = pl.program_id(1)
    xr = xyz_ref[...]
    xt = xyzt_ref[...]
    sr = jnp.sum(xr * xr, axis=1, keepdims=True)
    sc = jnp.sum(xt * xt, axis=0, keepdims=True)
    xr16 = xr.astype(_bf).astype(jnp.float32)
    xt16 = xt.astype(_bf).astype(jnp.float32)
    cross = (xr16[:, 0:1] * xt16[0:1, :]
             + xr16[:, 1:2] * xt16[1:2, :]
             + xr16[:, 2:3] * xt16[2:3, :])
    d = (sr + sc) - 2.0 * cross
    colio = lax.broadcasted_iota(jnp.int32, (tr, npts), 1)
    rowg = r * tr + lax.broadcasted_iota(jnp.int32, (tr, 1), 0)
    d = jnp.where(colio == rowg, 1e9, d)
    cols = []
    for _ in range(_K):
        mn = jnp.min(d, axis=1, keepdims=True)
        cand = jnp.where(d == mn, colio, npts)
        j = jnp.min(cand, axis=1, keepdims=True)
        cols.append(j)
        d = jnp.where(colio == j, 1e9, d)
    idx_ref[...] = jnp.concatenate(cols, axis=1)


def _knn(xyz, xyzt, npts):
    tr = 128
    return pl.pallas_call(
        _knn_body,
        grid=(_B, npts // tr),
        in_specs=[
            pl.BlockSpec((None, tr, 3), lambda b, r: (b, r, 0)),
            pl.BlockSpec((None, 3, npts), lambda b, r: (b, 0, 0)),
        ],
        out_specs=pl.BlockSpec((None, tr, _K), lambda b, r: (b, r, 0)),
        out_shape=jax.ShapeDtypeStruct((_B, npts, _K), jnp.int32),
    )(xyz, xyzt)


def _gather_feat(feat_tab, idx_flat):
    e_total = idx_flat.shape[0]
    width = feat_tab.shape[1]
    nw = 32
    ew = e_total // nw
    ch = 128 if ew % 256 == 0 else 112
    nch2 = ew // (2 * ch)
    mesh = plsc.VectorSubcoreMesh(core_axis_name="c", subcore_axis_name="s")

    def body(feat_hbm, idx_hbm, out_hbm, idxc0, idxc1, rows_v0, rows_v1,
             sem0, sem1):
        wid = lax.axis_index("s") * 2 + lax.axis_index("c")
        base = pl.multiple_of(wid * ew, 8)

        def step(ci, carry):
            off0 = pl.multiple_of(ci * 2 * ch, 8)
            off1 = pl.multiple_of(ci * 2 * ch + ch, 8)
            pltpu.sync_copy(idx_hbm.at[pl.ds(base + off0, ch)], idxc0)
            pltpu.sync_copy(idx_hbm.at[pl.ds(base + off1, ch)], idxc1)
            cp0 = pltpu.async_copy(feat_hbm.at[idxc0], rows_v0, sem0)
            cp1 = pltpu.async_copy(feat_hbm.at[idxc1], rows_v1, sem1)
            cp0.wait()
            pltpu.sync_copy(rows_v0, out_hbm.at[pl.ds(base + off0, ch)])
            cp1.wait()
            pltpu.sync_copy(rows_v1, out_hbm.at[pl.ds(base + off1, ch)])
            return carry

        lax.fori_loop(0, nch2, step, 0)

    gk = pl.kernel(
        body,
        out_type=jax.ShapeDtypeStruct((e_total, width), jnp.float32),
        mesh=mesh,
        scratch_types=[
            pltpu.VMEM((ch,), jnp.int32),
            pltpu.VMEM((ch,), jnp.int32),
            pltpu.VMEM((ch, width), jnp.float32),
            pltpu.VMEM((ch, width), jnp.float32),
            pltpu.SemaphoreType.DMA,
            pltpu.SemaphoreType.DMA,
        ],
    )
    return gk(feat_tab, idx_flat)


def _edge_body(feat_ref, xyz_ref, g_ref, w1tf_ref, w1tx_ref,
               w1bf_ref, w1bx_ref, b1_ref, w2_ref, b2_ref, cw1_ref, cb1_ref,
               cw2_ref, cb2_ref, feato_ref, xyzo_ref):
    feat = feat_ref[...]
    xyz = xyz_ref[...]
    t_xi = (_mm(feat, w1tf_ref[...]) + _mm(xyz, w1tx_ref[...])
            + b1_ref[...])
    w1bf = w1bf_ref[...]
    w1bx = w1bx_ref[...]
    w2 = w2_ref[...]
    m = jnp.full((feat.shape[0], _C), -1e30, jnp.float32)
    for k in range(_K):
        gk = g_ref[k]
        df = gk[:, :_C] - feat
        dx = gk[:, _C:_C + 3] - xyz
        h = jnp.maximum(t_xi + _mm(df, w1bf) + _mm(dx, w1bx), 0.0)
        s = _mm(h, w2)
        m = jnp.maximum(m, s)
    feat_o = m + b2_ref[...]
    feato_ref[...] = feat_o
    y = jnp.maximum(_mm(feat_o, cw1_ref[...]) + cb1_ref[...], 0.0)
    xyzo_ref[...] = xyz + _mm(y, cw2_ref[...]) + cb2_ref[...]


def _edge_mlp(feat, xyz, g, w1tf, w1tx, w1bf, w1bx, b1, w2, b2,
              cw1, cb1, cw2, cb2, npts, tre):
    return pl.pallas_call(
        _edge_body,
        grid=(_B, npts // tre),
        in_specs=[
            pl.BlockSpec((None, tre, _C), lambda b, t: (b, t, 0)),
            pl.BlockSpec((None, tre, 3), lambda b, t: (b, t, 0)),
            pl.BlockSpec((None, _K, tre, 256), lambda b, t: (b, 0, t, 0)),
            pl.BlockSpec((_C, _C), lambda b, t: (0, 0)),
            pl.BlockSpec((3, _C), lambda b, t: (0, 0)),
            pl.BlockSpec((_C, _C), lambda b, t: (0, 0)),
            pl.BlockSpec((3, _C), lambda b, t: (0, 0)),
            pl.BlockSpec((1, _C), lambda b, t: (0, 0)),
            pl.BlockSpec((_C, _C), lambda b, t: (0, 0)),
            pl.BlockSpec((1, _C), lambda b, t: (0, 0)),
            pl.BlockSpec((_C, _C), lambda b, t: (0, 0)),
            pl.BlockSpec((1, _C), lambda b, t: (0, 0)),
            pl.BlockSpec((_C, 3), lambda b, t: (0, 0)),
            pl.BlockSpec((1, 3), lambda b, t: (0, 0)),
        ],
        out_specs=[
            pl.BlockSpec((None, tre, _C), lambda b, t: (b, t, 0)),
            pl.BlockSpec((None, tre, 3), lambda b, t: (b, t, 0)),
        ],
        out_shape=[
            jax.ShapeDtypeStruct((_B, npts, _C), jnp.float32),
            jax.ShapeDtypeStruct((_B, npts, 3), jnp.float32),
        ],
    )(feat, xyz, g, w1tf, w1tx, w1bf, w1bx, b1.reshape(1, _C), w2,
      b2.reshape(1, _C), cw1, cb1.reshape(1, _C), cw2, cb2.reshape(1, 3))


def _edgeconv_big(xyz, feat, npts, w1tf, w1tx, w1bf, w1bx, b1, w2, b2,
                  cw1, cb1, cw2, cb2):
    tre = 512 if npts % 512 == 0 else 448
    xyzt = jnp.transpose(xyz, (0, 2, 1))
    idx = _knn(xyz, xyzt, npts)
    offs = (jnp.arange(_B, dtype=jnp.int32) * npts)[:, None, None]
    idx_flat = jnp.transpose(idx + offs, (0, 2, 1)).reshape(_B * _K * npts)
    nfp = jnp.concatenate(
        [feat, xyz, jnp.zeros((_B, npts, 125), jnp.float32)], axis=-1)
    g = _gather_feat(nfp.reshape(_B * npts, 256), idx_flat)
    g = g.reshape(_B, _K, npts, 256)
    return _edge_mlp(feat, xyz, g, w1tf, w1tx, w1bf, w1bx, b1, w2, b2,
                     cw1, cb1, cw2, cb2, npts, tre)


def _pred_body(xyz_ref, xyzt_ref, feat_ref, w1tf_ref, w1tx_ref, w1bx_ref,
               b1_ref, w2_ref, b2_ref, pw1_ref, pb1_ref, pw2_ref,
               pb2_ref, feato_ref, xyzo_ref):
    x = xyz_ref[...]
    xt = xyzt_ref[...]
    f = feat_ref[...]
    t1 = (_mm(f, w1tf_ref[...]) + _mm(x, w1tx_ref[...])
          + b1_ref[...])
    sqr = jnp.sum(x * x, axis=1, keepdims=True)
    sqc = jnp.sum(xt * xt, axis=0, keepdims=True)
    d = (sqr + sqc) - 2.0 * _mm(x, xt)
    colio = lax.broadcasted_iota(jnp.int32, (_PRED_R, _PRED_R), 1)
    rowio = lax.broadcasted_iota(jnp.int32, (_PRED_R, _PRED_R), 0)
    d = jnp.where(colio == rowio, 1e9, d)
    sel_mask = colio < 0
    for _ in range(_K):
        mn = jnp.min(d, axis=1, keepdims=True)
        cand = jnp.where(d == mn, colio, _PRED_R)
        j = jnp.min(cand, axis=1, keepdims=True)
        sel = colio == j
        sel_mask = jnp.logical_or(sel_mask, sel)
        d = jnp.where(sel, 1e9, d)
    w1bx = w1bx_ref[...]
    w2 = w2_ref[...]
    out = jnp.full((_PRED_R, _C), -1e30, jnp.float32)
    for j in range(_PRED_R):
        dx = x[j:j + 1, :] - x
        h = jnp.maximum(t1 + _mm(dx, w1bx), 0.0)
        s = _mm(h, w2)
        ok = sel_mask[:, j:j + 1]
        out = jnp.where(ok, jnp.maximum(out, s), out)
    feat_o = out + b2_ref[...]
    feato_ref[...] = feat_o
    y = jnp.maximum(_mm(feat_o, pw1_ref[...]) + pb1_ref[...], 0.0)
    xyzo_ref[...] = x + _mm(y, pw2_ref[...]) + pb2_ref[...]


def _pred_blocks(xyz_p, feat_p, w1tf, w1tx, w1bx, b1, w2, b2,
                 pw1, pb1, pw2, pb2):
    g = _B * _M
    xyz_g = xyz_p.reshape(g, _PRED_R, 3)
    xyzt_g = jnp.transpose(xyz_g, (0, 2, 1))
    feat_g = feat_p.reshape(g, 1, _C)
    wspec = lambda shp: pl.BlockSpec(shp, lambda i: tuple(0 for _ in shp))
    feat_o, xyz_o = pl.pallas_call(
        _pred_body,
        grid=(g,),
        in_specs=[
            pl.BlockSpec((None, _PRED_R, 3), lambda i: (i, 0, 0)),
            pl.BlockSpec((None, 3, _PRED_R), lambda i: (i, 0, 0)),
            pl.BlockSpec((None, 1, _C), lambda i: (i, 0, 0)),
            wspec((_C, _C)), wspec((3, _C)), wspec((3, _C)),
            wspec((1, _C)), wspec((_C, _C)), wspec((1, _C)),
            wspec((_C, _C)), wspec((1, _C)), wspec((_C, 3)), wspec((1, 3)),
        ],
        out_specs=[
            pl.BlockSpec((None, _PRED_R, _C), lambda i: (i, 0, 0)),
            pl.BlockSpec((None, _PRED_R, 3), lambda i: (i, 0, 0)),
        ],
        out_shape=[
            jax.ShapeDtypeStruct((g, _PRED_R, _C), jnp.float32),
            jax.ShapeDtypeStruct((g, _PRED_R, 3), jnp.float32),
        ],
    )(xyz_g, xyzt_g, feat_g, w1tf, w1tx, w1bx, b1.reshape(1, _C), w2,
      b2.reshape(1, _C), pw1, pb1.reshape(1, _C), pw2, pb2.reshape(1, 3))
    return feat_o.reshape(_B, _M, _PRED_R, _C), xyz_o.reshape(_B, _M, _PRED_R, 3)


def kernel(ctx_xyz, ctx_tokens, pred_tokens, fp_W, fp_b, ec_W1, ec_b1, ec_W2,
           ec_b2, co_W1, co_b1, co_W2, co_b2, po_W1, po_b1, po_W2, po_b2,
           fo_W1, fo_b1, fo_W2, fo_b2):
    nkey = jax.random.key(42)
    noise = jax.random.normal(jax.random.fold_in(nkey, 0),
                              (_B, _P, _UP_R, 3), dtype=jnp.float32)
    noise = noise / (jnp.linalg.norm(noise, axis=-1, keepdims=True) + 1e-6)
    noise = noise * _CTX_RADIUS
    xyz_ctx = (ctx_xyz[:, :, None, :] + noise).reshape(_B, _N_CTX, 3)

    ctx_feat = _feat_proj(ctx_tokens, fp_W, fp_b)
    feat_ctx = jnp.broadcast_to(
        ctx_feat[:, :, None, :], (_B, _P, _UP_R, _C)
    ).reshape(_B, _N_CTX, _C)

    in_dim = _C + 3
    w1tf = ec_W1[:_C]
    w1tx = ec_W1[_C:in_dim]
    w1bf = ec_W1[in_dim:in_dim + _C]
    w1bx = ec_W1[in_dim + _C:]

    feat1, xyz1 = _edgeconv_big(xyz_ctx, feat_ctx, _N_CTX, w1tf, w1tx, w1bf,
                                w1bx, ec_b1, ec_W2, ec_b2, co_W1, co_b1,
                                co_W2, co_b2)

    anchor = jnp.mean(ctx_xyz, axis=1)
    pns = []
    for m in range(_M):
        pn = jax.random.normal(jax.random.fold_in(nkey, 100 + m),
                               (_B, _PRED_R, 3), dtype=jnp.float32)
        pn = pn / (jnp.linalg.norm(pn, axis=-1, keepdims=True) + 1e-6)
        pns.append(pn * _PRED_RADIUS)
    xyz_p = anchor[:, None, None, :] + jnp.stack(pns, axis=1)
    feat_p, xyz_p_new = _pred_blocks(xyz_p, pred_tokens, w1tf, w1tx, w1bx,
                                     ec_b1, ec_W2, ec_b2, po_W1, po_b1,
                                     po_W2, po_b2)

    pad = _N_ALLP - _N_ALL
    sent = (1e4 + 10.0 * jnp.arange(pad, dtype=jnp.float32))
    sent = jnp.broadcast_to(sent[None, :, None], (_B, pad, 3))
    xyz_all = jnp.concatenate(
        [xyz1, xyz_p_new.reshape(_B, _M * _PRED_R, 3), sent], axis=1)
    feat_all = jnp.concatenate(
        [feat1, feat_p.reshape(_B, _M * _PRED_R, _C),
         jnp.zeros((_B, pad, _C), jnp.float32)], axis=1)

    _, xyz2 = _edgeconv_big(xyz_all, feat_all, _N_ALLP, w1tf, w1tx, w1bf,
                            w1bx, ec_b1, ec_W2, ec_b2, fo_W1, fo_b1, fo_W2,
                            fo_b2)
    return xyz2[:, :_N_ALL]


# speedup vs baseline: 6.6708x; 1.0431x over previous
import jax
import jax.numpy as jnp
from jax import lax
from jax.experimental import pallas as pl
from jax.experimental.pallas import tpu as pltpu
from jax.experimental.pallas import tpu_sc as plsc

_B, _P, _M = 2, 512, 4
_C = 128
_UP_R = 12
_PRED_R = 16
_CTX_RADIUS = 0.02
_PRED_RADIUS = 0.05
_K = 8
_N_CTX = _P * _UP_R
_N_ALL = _N_CTX + _M * _PRED_R
_N_ALLP = 6272

_bf = jnp.bfloat16


def _mm(a, b):
    return jnp.dot(a.astype(_bf), b.astype(_bf),
                   preferred_element_type=jnp.float32)


def _proj_body(tok_ref, w_ref, b_ref, out_ref):
    out_ref[...] = _mm(tok_ref[...], w_ref[...]) + b_ref[...]


def _feat_proj(ctx_tokens, fp_W, fp_b):
    d_in = ctx_tokens.shape[-1]
    return pl.pallas_call(
        _proj_body,
        grid=(_B,),
        in_specs=[
            pl.BlockSpec((None, _P, d_in), lambda b: (b, 0, 0)),
            pl.BlockSpec((d_in, _C), lambda b: (0, 0)),
            pl.BlockSpec((1, _C), lambda b: (0, 0)),
        ],
        out_specs=pl.BlockSpec((None, _P, _C), lambda b: (b, 0, 0)),
        out_shape=jax.ShapeDtypeStruct((_B, _P, _C), jnp.float32),
    )(ctx_tokens, fp_W, fp_b.reshape(1, _C))


def _knn_body(xyz_ref, xyzt_ref, idx_ref):
    tr = xyz_ref.shape[0]
    npts = xyzt_ref.shape[1]
    r = pl.program_id(1)
    xr = xyz_ref[...]
    xt = xyzt_ref[...]
    sr = jnp.sum(xr * xr, axis=1, keepdims=True)
    sc = jnp.sum(xt * xt, axis=0, keepdims=True)
    cross = _mm(xr, xt)
    d = (sr + sc) - 2.0 * cross
    colio = lax.broadcasted_iota(jnp.int32, (tr, npts), 1)
    rowg = r * tr + lax.broadcasted_iota(jnp.int32, (tr, 1), 0)
    d = jnp.where(colio == rowg, 1e9, d)
    cols = []
    for _ in range(_K):
        mn = jnp.min(d, axis=1, keepdims=True)
        cand = jnp.where(d == mn, colio, npts)
        j = jnp.min(cand, axis=1, keepdims=True)
        cols.append(j)
        d = jnp.where(colio == j, 1e9, d)
    idx_ref[...] = jnp.concatenate(cols, axis=1)


def _knn(xyz, xyzt, npts):
    tr = 128
    return pl.pallas_call(
        _knn_body,
        grid=(_B, npts // tr),
        in_specs=[
            pl.BlockSpec((None, tr, 3), lambda b, r: (b, r, 0)),
            pl.BlockSpec((None, 3, npts), lambda b, r: (b, 0, 0)),
        ],
        out_specs=pl.BlockSpec((None, tr, _K), lambda b, r: (b, r, 0)),
        out_shape=jax.ShapeDtypeStruct((_B, npts, _K), jnp.int32),
    )(xyz, xyzt)


def _gather_feat(feat_tab, idx_flat):
    e_total = idx_flat.shape[0]
    width = feat_tab.shape[1]
    nw = 32
    ew = e_total // nw
    ch = 128 if ew % 256 == 0 else 112
    nch2 = ew // (2 * ch)
    mesh = plsc.VectorSubcoreMesh(core_axis_name="c", subcore_axis_name="s")

    def body(feat_hbm, idx_hbm, out_hbm, idxc0, idxc1, rows_v0, rows_v1,
             sem0, sem1):
        wid = lax.axis_index("s") * 2 + lax.axis_index("c")
        base = pl.multiple_of(wid * ew, 8)

        def step(ci, carry):
            off0 = pl.multiple_of(ci * 2 * ch, 8)
            off1 = pl.multiple_of(ci * 2 * ch + ch, 8)
            pltpu.sync_copy(idx_hbm.at[pl.ds(base + off0, ch)], idxc0)
            pltpu.sync_copy(idx_hbm.at[pl.ds(base + off1, ch)], idxc1)
            cp0 = pltpu.async_copy(feat_hbm.at[idxc0], rows_v0, sem0)
            cp1 = pltpu.async_copy(feat_hbm.at[idxc1], rows_v1, sem1)
            cp0.wait()
            pltpu.sync_copy(rows_v0, out_hbm.at[pl.ds(base + off0, ch)])
            cp1.wait()
            pltpu.sync_copy(rows_v1, out_hbm.at[pl.ds(base + off1, ch)])
            return carry

        lax.fori_loop(0, nch2, step, 0)

    gk = pl.kernel(
        body,
        out_type=jax.ShapeDtypeStruct((e_total, width), jnp.float32),
        mesh=mesh,
        scratch_types=[
            pltpu.VMEM((ch,), jnp.int32),
            pltpu.VMEM((ch,), jnp.int32),
            pltpu.VMEM((ch, width), jnp.float32),
            pltpu.VMEM((ch, width), jnp.float32),
            pltpu.SemaphoreType.DMA,
            pltpu.SemaphoreType.DMA,
        ],
    )
    return gk(feat_tab, idx_flat)


def _edge_body(feat_ref, xyz_ref, g_ref, w1tf_ref, w1tx_ref,
               w1bf_ref, w1bx_ref, b1_ref, w2_ref, b2_ref, cw1_ref, cb1_ref,
               cw2_ref, cb2_ref, feato_ref, xyzo_ref):
    feat = feat_ref[...]
    xyz = xyz_ref[...]
    t_xi = (_mm(feat, w1tf_ref[...]) + _mm(xyz, w1tx_ref[...])
            + b1_ref[...])
    w1bf = w1bf_ref[...]
    w1bx = w1bx_ref[...]
    w2 = w2_ref[...]
    m = jnp.full((feat.shape[0], _C), -1e30, jnp.float32)
    for k in range(_K):
        gk = g_ref[k]
        df = gk[:, :_C] - feat
        dx = gk[:, _C:_C + 3] - xyz
        h = jnp.maximum(t_xi + _mm(df, w1bf) + _mm(dx, w1bx), 0.0)
        s = _mm(h, w2)
        m = jnp.maximum(m, s)
    feat_o = m + b2_ref[...]
    feato_ref[...] = feat_o
    y = jnp.maximum(_mm(feat_o, cw1_ref[...]) + cb1_ref[...], 0.0)
    xyzo_ref[...] = xyz + _mm(y, cw2_ref[...]) + cb2_ref[...]


def _edge_mlp(feat, xyz, g, w1tf, w1tx, w1bf, w1bx, b1, w2, b2,
              cw1, cb1, cw2, cb2, npts, tre):
    return pl.pallas_call(
        _edge_body,
        grid=(_B, npts // tre),
        in_specs=[
            pl.BlockSpec((None, tre, _C), lambda b, t: (b, t, 0)),
            pl.BlockSpec((None, tre, 3), lambda b, t: (b, t, 0)),
            pl.BlockSpec((None, _K, tre, 256), lambda b, t: (b, 0, t, 0)),
            pl.BlockSpec((_C, _C), lambda b, t: (0, 0)),
            pl.BlockSpec((3, _C), lambda b, t: (0, 0)),
            pl.BlockSpec((_C, _C), lambda b, t: (0, 0)),
            pl.BlockSpec((3, _C), lambda b, t: (0, 0)),
            pl.BlockSpec((1, _C), lambda b, t: (0, 0)),
            pl.BlockSpec((_C, _C), lambda b, t: (0, 0)),
            pl.BlockSpec((1, _C), lambda b, t: (0, 0)),
            pl.BlockSpec((_C, _C), lambda b, t: (0, 0)),
            pl.BlockSpec((1, _C), lambda b, t: (0, 0)),
            pl.BlockSpec((_C, 3), lambda b, t: (0, 0)),
            pl.BlockSpec((1, 3), lambda b, t: (0, 0)),
        ],
        out_specs=[
            pl.BlockSpec((None, tre, _C), lambda b, t: (b, t, 0)),
            pl.BlockSpec((None, tre, 3), lambda b, t: (b, t, 0)),
        ],
        out_shape=[
            jax.ShapeDtypeStruct((_B, npts, _C), jnp.float32),
            jax.ShapeDtypeStruct((_B, npts, 3), jnp.float32),
        ],
    )(feat, xyz, g, w1tf, w1tx, w1bf, w1bx, b1.reshape(1, _C), w2,
      b2.reshape(1, _C), cw1, cb1.reshape(1, _C), cw2, cb2.reshape(1, 3))


def _edgeconv_big(xyz, feat, npts, w1tf, w1tx, w1bf, w1bx, b1, w2, b2,
                  cw1, cb1, cw2, cb2):
    tre = 512 if npts % 512 == 0 else 448
    xyzt = jnp.transpose(xyz, (0, 2, 1))
    idx = _knn(xyz, xyzt, npts)
    offs = (jnp.arange(_B, dtype=jnp.int32) * npts)[:, None, None]
    idx_flat = jnp.transpose(idx + offs, (0, 2, 1)).reshape(_B * _K * npts)
    nfp = jnp.concatenate(
        [feat, xyz, jnp.zeros((_B, npts, 125), jnp.float32)], axis=-1)
    g = _gather_feat(nfp.reshape(_B * npts, 256), idx_flat)
    g = g.reshape(_B, _K, npts, 256)
    return _edge_mlp(feat, xyz, g, w1tf, w1tx, w1bf, w1bx, b1, w2, b2,
                     cw1, cb1, cw2, cb2, npts, tre)


def _pred_body(xyz_ref, xyzt_ref, feat_ref, w1tf_ref, w1tx_ref, w1bx_ref,
               b1_ref, w2_ref, b2_ref, pw1_ref, pb1_ref, pw2_ref,
               pb2_ref, feato_ref, xyzo_ref):
    x = xyz_ref[...]
    xt = xyzt_ref[...]
    f = feat_ref[...]
    t1 = (_mm(f, w1tf_ref[...]) + _mm(x, w1tx_ref[...])
          + b1_ref[...])
    sqr = jnp.sum(x * x, axis=1, keepdims=True)
    sqc = jnp.sum(xt * xt, axis=0, keepdims=True)
    d = (sqr + sqc) - 2.0 * _mm(x, xt)
    colio = lax.broadcasted_iota(jnp.int32, (_PRED_R, _PRED_R), 1)
    rowio = lax.broadcasted_iota(jnp.int32, (_PRED_R, _PRED_R), 0)
    d = jnp.where(colio == rowio, 1e9, d)
    sel_mask = colio < 0
    for _ in range(_K):
        mn = jnp.min(d, axis=1, keepdims=True)
        cand = jnp.where(d == mn, colio, _PRED_R)
        j = jnp.min(cand, axis=1, keepdims=True)
        sel = colio == j
        sel_mask = jnp.logical_or(sel_mask, sel)
        d = jnp.where(sel, 1e9, d)
    w1bx = w1bx_ref[...]
    w2 = w2_ref[...]
    out = jnp.full((_PRED_R, _C), -1e30, jnp.float32)
    for j in range(_PRED_R):
        dx = x[j:j + 1, :] - x
        h = jnp.maximum(t1 + _mm(dx, w1bx), 0.0)
        s = _mm(h, w2)
        ok = sel_mask[:, j:j + 1]
        out = jnp.where(ok, jnp.maximum(out, s), out)
    feat_o = out + b2_ref[...]
    feato_ref[...] = feat_o
    y = jnp.maximum(_mm(feat_o, pw1_ref[...]) + pb1_ref[...], 0.0)
    xyzo_ref[...] = x + _mm(y, pw2_ref[...]) + pb2_ref[...]


def _pred_blocks(xyz_p, feat_p, w1tf, w1tx, w1bx, b1, w2, b2,
                 pw1, pb1, pw2, pb2):
    g = _B * _M
    xyz_g = xyz_p.reshape(g, _PRED_R, 3)
    xyzt_g = jnp.transpose(xyz_g, (0, 2, 1))
    feat_g = feat_p.reshape(g, 1, _C)
    wspec = lambda shp: pl.BlockSpec(shp, lambda i: tuple(0 for _ in shp))
    feat_o, xyz_o = pl.pallas_call(
        _pred_body,
        grid=(g,),
        in_specs=[
            pl.BlockSpec((None, _PRED_R, 3), lambda i: (i, 0, 0)),
            pl.BlockSpec((None, 3, _PRED_R), lambda i: (i, 0, 0)),
            pl.BlockSpec((None, 1, _C), lambda i: (i, 0, 0)),
            wspec((_C, _C)), wspec((3, _C)), wspec((3, _C)),
            wspec((1, _C)), wspec((_C, _C)), wspec((1, _C)),
            wspec((_C, _C)), wspec((1, _C)), wspec((_C, 3)), wspec((1, 3)),
        ],
        out_specs=[
            pl.BlockSpec((None, _PRED_R, _C), lambda i: (i, 0, 0)),
            pl.BlockSpec((None, _PRED_R, 3), lambda i: (i, 0, 0)),
        ],
        out_shape=[
            jax.ShapeDtypeStruct((g, _PRED_R, _C), jnp.float32),
            jax.ShapeDtypeStruct((g, _PRED_R, 3), jnp.float32),
        ],
    )(xyz_g, xyzt_g, feat_g, w1tf, w1tx, w1bx, b1.reshape(1, _C), w2,
      b2.reshape(1, _C), pw1, pb1.reshape(1, _C), pw2, pb2.reshape(1, 3))
    return feat_o.reshape(_B, _M, _PRED_R, _C), xyz_o.reshape(_B, _M, _PRED_R, 3)


def kernel(ctx_xyz, ctx_tokens, pred_tokens, fp_W, fp_b, ec_W1, ec_b1, ec_W2,
           ec_b2, co_W1, co_b1, co_W2, co_b2, po_W1, po_b1, po_W2, po_b2,
           fo_W1, fo_b1, fo_W2, fo_b2):
    nkey = jax.random.key(42)
    noise = jax.random.normal(jax.random.fold_in(nkey, 0),
                              (_B, _P, _UP_R, 3), dtype=jnp.float32)
    noise = noise / (jnp.linalg.norm(noise, axis=-1, keepdims=True) + 1e-6)
    noise = noise * _CTX_RADIUS
    xyz_ctx = (ctx_xyz[:, :, None, :] + noise).reshape(_B, _N_CTX, 3)

    ctx_feat = _feat_proj(ctx_tokens, fp_W, fp_b)
    feat_ctx = jnp.broadcast_to(
        ctx_feat[:, :, None, :], (_B, _P, _UP_R, _C)
    ).reshape(_B, _N_CTX, _C)

    in_dim = _C + 3
    w1tf = ec_W1[:_C]
    w1tx = ec_W1[_C:in_dim]
    w1bf = ec_W1[in_dim:in_dim + _C]
    w1bx = ec_W1[in_dim + _C:]

    feat1, xyz1 = _edgeconv_big(xyz_ctx, feat_ctx, _N_CTX, w1tf, w1tx, w1bf,
                                w1bx, ec_b1, ec_W2, ec_b2, co_W1, co_b1,
                                co_W2, co_b2)

    anchor = jnp.mean(ctx_xyz, axis=1)
    pns = []
    for m in range(_M):
        pn = jax.random.normal(jax.random.fold_in(nkey, 100 + m),
                               (_B, _PRED_R, 3), dtype=jnp.float32)
        pn = pn / (jnp.linalg.norm(pn, axis=-1, keepdims=True) + 1e-6)
        pns.append(pn * _PRED_RADIUS)
    xyz_p = anchor[:, None, None, :] + jnp.stack(pns, axis=1)
    feat_p, xyz_p_new = _pred_blocks(xyz_p, pred_tokens, w1tf, w1tx, w1bx,
                                     ec_b1, ec_W2, ec_b2, po_W1, po_b1,
                                     po_W2, po_b2)

    pad = _N_ALLP - _N_ALL
    sent = (1e4 + 10.0 * jnp.arange(pad, dtype=jnp.float32))
    sent = jnp.broadcast_to(sent[None, :, None], (_B, pad, 3))
    xyz_all = jnp.concatenate(
        [xyz1, xyz_p_new.reshape(_B, _M * _PRED_R, 3), sent], axis=1)
    feat_all = jnp.concatenate(
        [feat1, feat_p.reshape(_B, _M * _PRED_R, _C),
         jnp.zeros((_B, pad, _C), jnp.float32)], axis=1)

    _, xyz2 = _edgeconv_big(xyz_all, feat_all, _N_ALLP, w1tf, w1tx, w1bf,
                            w1bx, ec_b1, ec_W2, ec_b2, fo_W1, fo_b1, fo_W2,
                            fo_b2)
    return xyz2[:, :_N_ALL]


# speedup vs baseline: 7.1198x; 1.0673x over previous
import jax
import jax.numpy as jnp
from jax import lax
from jax.experimental import pallas as pl
from jax.experimental.pallas import tpu as pltpu
from jax.experimental.pallas import tpu_sc as plsc

_B, _P, _M = 2, 512, 4
_C = 128
_UP_R = 12
_PRED_R = 16
_CTX_RADIUS = 0.02
_PRED_RADIUS = 0.05
_K = 8
_N_CTX = _P * _UP_R
_N_ALL = _N_CTX + _M * _PRED_R
_N_ALLP = 6272

_bf = jnp.bfloat16


def _mm(a, b):
    return jnp.dot(a.astype(_bf), b.astype(_bf),
                   preferred_element_type=jnp.float32)


def _proj_body(tok_ref, w_ref, b_ref, out_ref):
    out_ref[...] = _mm(tok_ref[...], w_ref[...]) + b_ref[...]


def _feat_proj(ctx_tokens, fp_W, fp_b):
    d_in = ctx_tokens.shape[-1]
    return pl.pallas_call(
        _proj_body,
        grid=(_B,),
        in_specs=[
            pl.BlockSpec((None, _P, d_in), lambda b: (b, 0, 0)),
            pl.BlockSpec((d_in, _C), lambda b: (0, 0)),
            pl.BlockSpec((1, _C), lambda b: (0, 0)),
        ],
        out_specs=pl.BlockSpec((None, _P, _C), lambda b: (b, 0, 0)),
        out_shape=jax.ShapeDtypeStruct((_B, _P, _C), jnp.float32),
    )(ctx_tokens, fp_W, fp_b.reshape(1, _C))


def _knn_body(xyz_ref, xyzt_ref, idx_ref):
    tr = xyz_ref.shape[0]
    npts = xyzt_ref.shape[1]
    r = pl.program_id(1)
    xr = xyz_ref[...]
    xt = xyzt_ref[...]
    sr = jnp.sum(xr * xr, axis=1, keepdims=True)
    sc = jnp.sum(xt * xt, axis=0, keepdims=True)
    cross = _mm(xr, xt)
    d = (sr + sc) - 2.0 * cross
    colio = lax.broadcasted_iota(jnp.int32, (tr, npts), 1)
    rowg = r * tr + lax.broadcasted_iota(jnp.int32, (tr, 1), 0)
    d = jnp.where(colio == rowg, 1e9, d)
    cols = []
    for _ in range(_K):
        mn = jnp.min(d, axis=1, keepdims=True)
        cand = jnp.where(d == mn, colio, npts)
        j = jnp.min(cand, axis=1, keepdims=True)
        cols.append(j)
        d = jnp.where(colio == j, 1e9, d)
    idx_ref[...] = jnp.concatenate(cols, axis=1)


def _knn(xyz, xyzt, npts):
    tr = 128
    return pl.pallas_call(
        _knn_body,
        grid=(xyz.shape[0], npts // tr),
        in_specs=[
            pl.BlockSpec((None, tr, 3), lambda b, r: (b, r, 0)),
            pl.BlockSpec((None, 3, npts), lambda b, r: (b, 0, 0)),
        ],
        out_specs=pl.BlockSpec((None, tr, _K), lambda b, r: (b, r, 0)),
        out_shape=jax.ShapeDtypeStruct((xyz.shape[0], npts, _K), jnp.int32),
    )(xyz, xyzt)


def _gather_feat(feat_tab, idx_flat):
    e_total = idx_flat.shape[0]
    width = feat_tab.shape[1]
    nw = 32
    ew = e_total // nw
    ch = 128 if ew % 256 == 0 else 112
    nch2 = ew // (2 * ch)
    mesh = plsc.VectorSubcoreMesh(core_axis_name="c", subcore_axis_name="s")

    def body(feat_hbm, idx_hbm, out_hbm, idxc0, idxc1, rows_v0, rows_v1,
             sem0, sem1):
        wid = lax.axis_index("s") * 2 + lax.axis_index("c")
        base = pl.multiple_of(wid * ew, 8)

        def step(ci, carry):
            off0 = pl.multiple_of(ci * 2 * ch, 8)
            off1 = pl.multiple_of(ci * 2 * ch + ch, 8)
            pltpu.sync_copy(idx_hbm.at[pl.ds(base + off0, ch)], idxc0)
            pltpu.sync_copy(idx_hbm.at[pl.ds(base + off1, ch)], idxc1)
            cp0 = pltpu.async_copy(feat_hbm.at[idxc0], rows_v0, sem0)
            cp1 = pltpu.async_copy(feat_hbm.at[idxc1], rows_v1, sem1)
            cp0.wait()
            pltpu.sync_copy(rows_v0, out_hbm.at[pl.ds(base + off0, ch)])
            cp1.wait()
            pltpu.sync_copy(rows_v1, out_hbm.at[pl.ds(base + off1, ch)])
            return carry

        lax.fori_loop(0, nch2, step, 0)

    gk = pl.kernel(
        body,
        out_type=jax.ShapeDtypeStruct((e_total, width), jnp.float32),
        mesh=mesh,
        scratch_types=[
            pltpu.VMEM((ch,), jnp.int32),
            pltpu.VMEM((ch,), jnp.int32),
            pltpu.VMEM((ch, width), jnp.float32),
            pltpu.VMEM((ch, width), jnp.float32),
            pltpu.SemaphoreType.DMA,
            pltpu.SemaphoreType.DMA,
        ],
    )
    return gk(feat_tab, idx_flat)


def _edge_body(feat_ref, xyz_ref, g_ref, w1tf_ref, w1tx_ref,
               w1bf_ref, w1bx_ref, b1_ref, w2_ref, b2_ref, cw1_ref, cb1_ref,
               cw2_ref, cb2_ref, feato_ref, xyzo_ref):
    feat = feat_ref[...]
    xyz = xyz_ref[...]
    t_xi = (_mm(feat, w1tf_ref[...]) + _mm(xyz, w1tx_ref[...])
            + b1_ref[...])
    w1bf = w1bf_ref[...]
    w1bx = w1bx_ref[...]
    w2 = w2_ref[...]
    m = jnp.full((feat.shape[0], _C), -1e30, jnp.float32)
    for k in range(_K):
        gk = g_ref[k]
        df = gk[:, :_C] - feat
        dx = gk[:, _C:_C + 3] - xyz
        h = jnp.maximum(t_xi + _mm(df, w1bf) + _mm(dx, w1bx), 0.0)
        s = _mm(h, w2)
        m = jnp.maximum(m, s)
    feat_o = m + b2_ref[...]
    feato_ref[...] = feat_o
    y = jnp.maximum(_mm(feat_o, cw1_ref[...]) + cb1_ref[...], 0.0)
    xyzo_ref[...] = xyz + _mm(y, cw2_ref[...]) + cb2_ref[...]


def _edge_mlp(feat, xyz, g, w1tf, w1tx, w1bf, w1bx, b1, w2, b2,
              cw1, cb1, cw2, cb2, npts, tre):
    return pl.pallas_call(
        _edge_body,
        grid=(feat.shape[0], npts // tre),
        in_specs=[
            pl.BlockSpec((None, tre, _C), lambda b, t: (b, t, 0)),
            pl.BlockSpec((None, tre, 3), lambda b, t: (b, t, 0)),
            pl.BlockSpec((None, _K, tre, 256), lambda b, t: (b, 0, t, 0)),
            pl.BlockSpec((_C, _C), lambda b, t: (0, 0)),
            pl.BlockSpec((3, _C), lambda b, t: (0, 0)),
            pl.BlockSpec((_C, _C), lambda b, t: (0, 0)),
            pl.BlockSpec((3, _C), lambda b, t: (0, 0)),
            pl.BlockSpec((1, _C), lambda b, t: (0, 0)),
            pl.BlockSpec((_C, _C), lambda b, t: (0, 0)),
            pl.BlockSpec((1, _C), lambda b, t: (0, 0)),
            pl.BlockSpec((_C, _C), lambda b, t: (0, 0)),
            pl.BlockSpec((1, _C), lambda b, t: (0, 0)),
            pl.BlockSpec((_C, 3), lambda b, t: (0, 0)),
            pl.BlockSpec((1, 3), lambda b, t: (0, 0)),
        ],
        out_specs=[
            pl.BlockSpec((None, tre, _C), lambda b, t: (b, t, 0)),
            pl.BlockSpec((None, tre, 3), lambda b, t: (b, t, 0)),
        ],
        out_shape=[
            jax.ShapeDtypeStruct((feat.shape[0], npts, _C), jnp.float32),
            jax.ShapeDtypeStruct((feat.shape[0], npts, 3), jnp.float32),
        ],
    )(feat, xyz, g, w1tf, w1tx, w1bf, w1bx, b1.reshape(1, _C), w2,
      b2.reshape(1, _C), cw1, cb1.reshape(1, _C), cw2, cb2.reshape(1, 3))


def _edgeconv_big(xyz, feat, npts, w1tf, w1tx, w1bf, w1bx, b1, w2, b2,
                  cw1, cb1, cw2, cb2):
    tre = 512 if npts % 512 == 0 else 448
    xyzt = jnp.transpose(xyz, (0, 2, 1))
    nfp = jnp.concatenate(
        [feat, xyz, jnp.zeros((_B, npts, 125), jnp.float32)], axis=-1)
    idxs = [_knn(xyz[b:b + 1], xyzt[b:b + 1], npts) for b in range(_B)]
    feat_os, xyz_os = [], []
    for b in range(_B):
        idx_flat = jnp.transpose(idxs[b][0], (1, 0)).reshape(_K * npts)
        g = _gather_feat(nfp[b], idx_flat).reshape(1, _K, npts, 256)
        f_o, x_o = _edge_mlp(feat[b:b + 1], xyz[b:b + 1], g, w1tf, w1tx,
                             w1bf, w1bx, b1, w2, b2, cw1, cb1, cw2, cb2,
                             npts, tre)
        feat_os.append(f_o)
        xyz_os.append(x_o)
    return (jnp.concatenate(feat_os, axis=0),
            jnp.concatenate(xyz_os, axis=0))


def _pred_body(xyz_ref, xyzt_ref, feat_ref, w1tf_ref, w1tx_ref, w1bx_ref,
               b1_ref, w2_ref, b2_ref, pw1_ref, pb1_ref, pw2_ref,
               pb2_ref, feato_ref, xyzo_ref):
    x = xyz_ref[...]
    xt = xyzt_ref[...]
    f = feat_ref[...]
    t1 = (_mm(f, w1tf_ref[...]) + _mm(x, w1tx_ref[...])
          + b1_ref[...])
    sqr = jnp.sum(x * x, axis=1, keepdims=True)
    sqc = jnp.sum(xt * xt, axis=0, keepdims=True)
    d = (sqr + sqc) - 2.0 * _mm(x, xt)
    colio = lax.broadcasted_iota(jnp.int32, (_PRED_R, _PRED_R), 1)
    rowio = lax.broadcasted_iota(jnp.int32, (_PRED_R, _PRED_R), 0)
    d = jnp.where(colio == rowio, 1e9, d)
    sel_mask = colio < 0
    for _ in range(_K):
        mn = jnp.min(d, axis=1, keepdims=True)
        cand = jnp.where(d == mn, colio, _PRED_R)
        j = jnp.min(cand, axis=1, keepdims=True)
        sel = colio == j
        sel_mask = jnp.logical_or(sel_mask, sel)
        d = jnp.where(sel, 1e9, d)
    w1bx = w1bx_ref[...]
    w2 = w2_ref[...]
    out = jnp.full((_PRED_R, _C), -1e30, jnp.float32)
    for j in range(_PRED_R):
        dx = x[j:j + 1, :] - x
        h = jnp.maximum(t1 + _mm(dx, w1bx), 0.0)
        s = _mm(h, w2)
        ok = sel_mask[:, j:j + 1]
        out = jnp.where(ok, jnp.maximum(out, s), out)
    feat_o = out + b2_ref[...]
    feato_ref[...] = feat_o
    y = jnp.maximum(_mm(feat_o, pw1_ref[...]) + pb1_ref[...], 0.0)
    xyzo_ref[...] = x + _mm(y, pw2_ref[...]) + pb2_ref[...]


def _pred_blocks(xyz_p, feat_p, w1tf, w1tx, w1bx, b1, w2, b2,
                 pw1, pb1, pw2, pb2):
    g = _B * _M
    xyz_g = xyz_p.reshape(g, _PRED_R, 3)
    xyzt_g = jnp.transpose(xyz_g, (0, 2, 1))
    feat_g = feat_p.reshape(g, 1, _C)
    wspec = lambda shp: pl.BlockSpec(shp, lambda i: tuple(0 for _ in shp))
    feat_o, xyz_o = pl.pallas_call(
        _pred_body,
        grid=(g,),
        in_specs=[
            pl.BlockSpec((None, _PRED_R, 3), lambda i: (i, 0, 0)),
            pl.BlockSpec((None, 3, _PRED_R), lambda i: (i, 0, 0)),
            pl.BlockSpec((None, 1, _C), lambda i: (i, 0, 0)),
            wspec((_C, _C)), wspec((3, _C)), wspec((3, _C)),
            wspec((1, _C)), wspec((_C, _C)), wspec((1, _C)),
            wspec((_C, _C)), wspec((1, _C)), wspec((_C, 3)), wspec((1, 3)),
        ],
        out_specs=[
            pl.BlockSpec((None, _PRED_R, _C), lambda i: (i, 0, 0)),
            pl.BlockSpec((None, _PRED_R, 3), lambda i: (i, 0, 0)),
        ],
        out_shape=[
            jax.ShapeDtypeStruct((g, _PRED_R, _C), jnp.float32),
            jax.ShapeDtypeStruct((g, _PRED_R, 3), jnp.float32),
        ],
    )(xyz_g, xyzt_g, feat_g, w1tf, w1tx, w1bx, b1.reshape(1, _C), w2,
      b2.reshape(1, _C), pw1, pb1.reshape(1, _C), pw2, pb2.reshape(1, 3))
    return feat_o.reshape(_B, _M, _PRED_R, _C), xyz_o.reshape(_B, _M, _PRED_R, 3)


def kernel(ctx_xyz, ctx_tokens, pred_tokens, fp_W, fp_b, ec_W1, ec_b1, ec_W2,
           ec_b2, co_W1, co_b1, co_W2, co_b2, po_W1, po_b1, po_W2, po_b2,
           fo_W1, fo_b1, fo_W2, fo_b2):
    nkey = jax.random.key(42)
    noise = jax.random.normal(jax.random.fold_in(nkey, 0),
                              (_B, _P, _UP_R, 3), dtype=jnp.float32)
    noise = noise / (jnp.linalg.norm(noise, axis=-1, keepdims=True) + 1e-6)
    noise = noise * _CTX_RADIUS
    xyz_ctx = (ctx_xyz[:, :, None, :] + noise).reshape(_B, _N_CTX, 3)

    ctx_feat = _feat_proj(ctx_tokens, fp_W, fp_b)
    feat_ctx = jnp.broadcast_to(
        ctx_feat[:, :, None, :], (_B, _P, _UP_R, _C)
    ).reshape(_B, _N_CTX, _C)

    in_dim = _C + 3
    w1tf = ec_W1[:_C]
    w1tx = ec_W1[_C:in_dim]
    w1bf = ec_W1[in_dim:in_dim + _C]
    w1bx = ec_W1[in_dim + _C:]

    feat1, xyz1 = _edgeconv_big(xyz_ctx, feat_ctx, _N_CTX, w1tf, w1tx, w1bf,
                                w1bx, ec_b1, ec_W2, ec_b2, co_W1, co_b1,
                                co_W2, co_b2)

    anchor = jnp.mean(ctx_xyz, axis=1)
    pns = []
    for m in range(_M):
        pn = jax.random.normal(jax.random.fold_in(nkey, 100 + m),
                               (_B, _PRED_R, 3), dtype=jnp.float32)
        pn = pn / (jnp.linalg.norm(pn, axis=-1, keepdims=True) + 1e-6)
        pns.append(pn * _PRED_RADIUS)
    xyz_p = anchor[:, None, None, :] + jnp.stack(pns, axis=1)
    feat_p, xyz_p_new = _pred_blocks(xyz_p, pred_tokens, w1tf, w1tx, w1bx,
                                     ec_b1, ec_W2, ec_b2, po_W1, po_b1,
                                     po_W2, po_b2)

    pad = _N_ALLP - _N_ALL
    sent = (1e4 + 10.0 * jnp.arange(pad, dtype=jnp.float32))
    sent = jnp.broadcast_to(sent[None, :, None], (_B, pad, 3))
    xyz_all = jnp.concatenate(
        [xyz1, xyz_p_new.reshape(_B, _M * _PRED_R, 3), sent], axis=1)
    feat_all = jnp.concatenate(
        [feat1, feat_p.reshape(_B, _M * _PRED_R, _C),
         jnp.zeros((_B, pad, _C), jnp.float32)], axis=1)

    _, xyz2 = _edgeconv_big(xyz_all, feat_all, _N_ALLP, w1tf, w1tx, w1bf,
                            w1bx, ec_b1, ec_W2, ec_b2, fo_W1, fo_b1, fo_W2,
                            fo_b2)
    return xyz2[:, :_N_ALL]


# speedup vs baseline: 7.4914x; 1.0522x over previous
import jax
import jax.numpy as jnp
from jax import lax
from jax.experimental import pallas as pl
from jax.experimental.pallas import tpu as pltpu
from jax.experimental.pallas import tpu_sc as plsc

_B, _P, _M = 2, 512, 4
_C = 128
_UP_R = 12
_PRED_R = 16
_CTX_RADIUS = 0.02
_PRED_RADIUS = 0.05
_K = 8
_N_CTX = _P * _UP_R
_N_ALL = _N_CTX + _M * _PRED_R
_N_ALLP = 6272

_bf = jnp.bfloat16


def _mm(a, b):
    return jnp.dot(a.astype(_bf), b.astype(_bf),
                   preferred_element_type=jnp.float32)


def _proj_body(tok_ref, w_ref, b_ref, out_ref):
    out_ref[...] = _mm(tok_ref[...], w_ref[...]) + b_ref[...]


def _feat_proj(ctx_tokens, fp_W, fp_b):
    d_in = ctx_tokens.shape[-1]
    return pl.pallas_call(
        _proj_body,
        grid=(_B,),
        in_specs=[
            pl.BlockSpec((None, _P, d_in), lambda b: (b, 0, 0)),
            pl.BlockSpec((d_in, _C), lambda b: (0, 0)),
            pl.BlockSpec((1, _C), lambda b: (0, 0)),
        ],
        out_specs=pl.BlockSpec((None, _P, _C), lambda b: (b, 0, 0)),
        out_shape=jax.ShapeDtypeStruct((_B, _P, _C), jnp.float32),
    )(ctx_tokens, fp_W, fp_b.reshape(1, _C))


def _knn_body(xyz_ref, xyzt_ref, idx_ref):
    tr = xyz_ref.shape[0]
    npts = xyzt_ref.shape[1]
    r = pl.program_id(1)
    xr = xyz_ref[...]
    xt = xyzt_ref[...]
    sr = jnp.sum(xr * xr, axis=1, keepdims=True)
    sc = jnp.sum(xt * xt, axis=0, keepdims=True)
    cross = _mm(xr, xt)
    d = (sr + sc) - 2.0 * cross
    colio = lax.broadcasted_iota(jnp.int32, (tr, npts), 1)
    rowg = r * tr + lax.broadcasted_iota(jnp.int32, (tr, 1), 0)
    d = jnp.where(colio == rowg, 1e9, d)
    cols = []
    for _ in range(_K):
        mn = jnp.min(d, axis=1, keepdims=True)
        cand = jnp.where(d == mn, colio, npts)
        j = jnp.min(cand, axis=1, keepdims=True)
        cols.append(j)
        d = jnp.where(colio == j, 1e9, d)
    idx_ref[...] = jnp.concatenate(cols, axis=1)


def _knn(xyz, xyzt, npts):
    tr = 192 if npts % 192 == 0 else 224
    return pl.pallas_call(
        _knn_body,
        grid=(xyz.shape[0], npts // tr),
        in_specs=[
            pl.BlockSpec((None, tr, 3), lambda b, r: (b, r, 0)),
            pl.BlockSpec((None, 3, npts), lambda b, r: (b, 0, 0)),
        ],
        out_specs=pl.BlockSpec((None, tr, _K), lambda b, r: (b, r, 0)),
        out_shape=jax.ShapeDtypeStruct((xyz.shape[0], npts, _K), jnp.int32),
    )(xyz, xyzt)


def _gather_feat(feat_tab, idx_flat):
    e_total = idx_flat.shape[0]
    width = feat_tab.shape[1]
    nw = 32
    ew = e_total // nw
    ch = 128 if ew % 256 == 0 else 112
    nch2 = ew // (2 * ch)
    mesh = plsc.VectorSubcoreMesh(core_axis_name="c", subcore_axis_name="s")

    def body(feat_hbm, idx_hbm, out_hbm, idxc0, idxc1, rows_v0, rows_v1,
             sem0, sem1):
        wid = lax.axis_index("s") * 2 + lax.axis_index("c")
        base = pl.multiple_of(wid * ew, 8)

        def step(ci, carry):
            off0 = pl.multiple_of(ci * 2 * ch, 8)
            off1 = pl.multiple_of(ci * 2 * ch + ch, 8)
            pltpu.sync_copy(idx_hbm.at[pl.ds(base + off0, ch)], idxc0)
            pltpu.sync_copy(idx_hbm.at[pl.ds(base + off1, ch)], idxc1)
            cp0 = pltpu.async_copy(feat_hbm.at[idxc0], rows_v0, sem0)
            cp1 = pltpu.async_copy(feat_hbm.at[idxc1], rows_v1, sem1)
            cp0.wait()
            pltpu.sync_copy(rows_v0, out_hbm.at[pl.ds(base + off0, ch)])
            cp1.wait()
            pltpu.sync_copy(rows_v1, out_hbm.at[pl.ds(base + off1, ch)])
            return carry

        lax.fori_loop(0, nch2, step, 0)

    gk = pl.kernel(
        body,
        out_type=jax.ShapeDtypeStruct((e_total, width), jnp.float32),
        mesh=mesh,
        scratch_types=[
            pltpu.VMEM((ch,), jnp.int32),
            pltpu.VMEM((ch,), jnp.int32),
            pltpu.VMEM((ch, width), jnp.float32),
            pltpu.VMEM((ch, width), jnp.float32),
            pltpu.SemaphoreType.DMA,
            pltpu.SemaphoreType.DMA,
        ],
    )
    return gk(feat_tab, idx_flat)


def _edge_body(feat_ref, xyz_ref, g_ref, w1tf_ref, w1tx_ref,
               w1bf_ref, w1bx_ref, b1_ref, w2_ref, b2_ref, cw1_ref, cb1_ref,
               cw2_ref, cb2_ref, feato_ref, xyzo_ref):
    feat = feat_ref[...]
    xyz = xyz_ref[...]
    t_xi = (_mm(feat, w1tf_ref[...]) + _mm(xyz, w1tx_ref[...])
            + b1_ref[...])
    w1bf = w1bf_ref[...]
    w1bx = w1bx_ref[...]
    w2 = w2_ref[...]
    m = jnp.full((feat.shape[0], _C), -1e30, jnp.float32)
    for k in range(_K):
        gk = g_ref[k]
        df = gk[:, :_C] - feat
        dx = gk[:, _C:_C + 3] - xyz
        h = jnp.maximum(t_xi + _mm(df, w1bf) + _mm(dx, w1bx), 0.0)
        s = _mm(h, w2)
        m = jnp.maximum(m, s)
    feat_o = m + b2_ref[...]
    feato_ref[...] = feat_o
    y = jnp.maximum(_mm(feat_o, cw1_ref[...]) + cb1_ref[...], 0.0)
    xyzo_ref[...] = xyz + _mm(y, cw2_ref[...]) + cb2_ref[...]


def _edge_mlp(feat, xyz, g, w1tf, w1tx, w1bf, w1bx, b1, w2, b2,
              cw1, cb1, cw2, cb2, npts, tre):
    return pl.pallas_call(
        _edge_body,
        grid=(feat.shape[0], npts // tre),
        in_specs=[
            pl.BlockSpec((None, tre, _C), lambda b, t: (b, t, 0)),
            pl.BlockSpec((None, tre, 3), lambda b, t: (b, t, 0)),
            pl.BlockSpec((None, _K, tre, 256), lambda b, t: (b, 0, t, 0)),
            pl.BlockSpec((_C, _C), lambda b, t: (0, 0)),
            pl.BlockSpec((3, _C), lambda b, t: (0, 0)),
            pl.BlockSpec((_C, _C), lambda b, t: (0, 0)),
            pl.BlockSpec((3, _C), lambda b, t: (0, 0)),
            pl.BlockSpec((1, _C), lambda b, t: (0, 0)),
            pl.BlockSpec((_C, _C), lambda b, t: (0, 0)),
            pl.BlockSpec((1, _C), lambda b, t: (0, 0)),
            pl.BlockSpec((_C, _C), lambda b, t: (0, 0)),
            pl.BlockSpec((1, _C), lambda b, t: (0, 0)),
            pl.BlockSpec((_C, 3), lambda b, t: (0, 0)),
            pl.BlockSpec((1, 3), lambda b, t: (0, 0)),
        ],
        out_specs=[
            pl.BlockSpec((None, tre, _C), lambda b, t: (b, t, 0)),
            pl.BlockSpec((None, tre, 3), lambda b, t: (b, t, 0)),
        ],
        out_shape=[
            jax.ShapeDtypeStruct((feat.shape[0], npts, _C), jnp.float32),
            jax.ShapeDtypeStruct((feat.shape[0], npts, 3), jnp.float32),
        ],
    )(feat, xyz, g, w1tf, w1tx, w1bf, w1bx, b1.reshape(1, _C), w2,
      b2.reshape(1, _C), cw1, cb1.reshape(1, _C), cw2, cb2.reshape(1, 3))


def _edgeconv_big(xyz, feat, npts, w1tf, w1tx, w1bf, w1bx, b1, w2, b2,
                  cw1, cb1, cw2, cb2):
    tre = 512 if npts % 512 == 0 else 448
    xyzt = jnp.transpose(xyz, (0, 2, 1))
    nfp = jnp.concatenate(
        [feat, xyz, jnp.zeros((_B, npts, 125), jnp.float32)], axis=-1)
    idxs = [_knn(xyz[b:b + 1], xyzt[b:b + 1], npts) for b in range(_B)]
    feat_os, xyz_os = [], []
    for b in range(_B):
        idx_flat = jnp.transpose(idxs[b][0], (1, 0)).reshape(_K * npts)
        g = _gather_feat(nfp[b], idx_flat).reshape(1, _K, npts, 256)
        f_o, x_o = _edge_mlp(feat[b:b + 1], xyz[b:b + 1], g, w1tf, w1tx,
                             w1bf, w1bx, b1, w2, b2, cw1, cb1, cw2, cb2,
                             npts, tre)
        feat_os.append(f_o)
        xyz_os.append(x_o)
    return (jnp.concatenate(feat_os, axis=0),
            jnp.concatenate(xyz_os, axis=0))


def _pred_body(xyz_ref, xyzt_ref, feat_ref, w1tf_ref, w1tx_ref, w1bx_ref,
               b1_ref, w2_ref, b2_ref, pw1_ref, pb1_ref, pw2_ref,
               pb2_ref, feato_ref, xyzo_ref):
    x = xyz_ref[...]
    xt = xyzt_ref[...]
    f = feat_ref[...]
    t1 = (_mm(f, w1tf_ref[...]) + _mm(x, w1tx_ref[...])
          + b1_ref[...])
    sqr = jnp.sum(x * x, axis=1, keepdims=True)
    sqc = jnp.sum(xt * xt, axis=0, keepdims=True)
    d = (sqr + sqc) - 2.0 * _mm(x, xt)
    colio = lax.broadcasted_iota(jnp.int32, (_PRED_R, _PRED_R), 1)
    rowio = lax.broadcasted_iota(jnp.int32, (_PRED_R, _PRED_R), 0)
    d = jnp.where(colio == rowio, 1e9, d)
    sel_mask = colio < 0
    for _ in range(_K):
        mn = jnp.min(d, axis=1, keepdims=True)
        cand = jnp.where(d == mn, colio, _PRED_R)
        j = jnp.min(cand, axis=1, keepdims=True)
        sel = colio == j
        sel_mask = jnp.logical_or(sel_mask, sel)
        d = jnp.where(sel, 1e9, d)
    w1bx = w1bx_ref[...]
    w2 = w2_ref[...]
    out = jnp.full((_PRED_R, _C), -1e30, jnp.float32)
    for j in range(_PRED_R):
        dx = x[j:j + 1, :] - x
        h = jnp.maximum(t1 + _mm(dx, w1bx), 0.0)
        s = _mm(h, w2)
        ok = sel_mask[:, j:j + 1]
        out = jnp.where(ok, jnp.maximum(out, s), out)
    feat_o = out + b2_ref[...]
    feato_ref[...] = feat_o
    y = jnp.maximum(_mm(feat_o, pw1_ref[...]) + pb1_ref[...], 0.0)
    xyzo_ref[...] = x + _mm(y, pw2_ref[...]) + pb2_ref[...]


def _pred_blocks(xyz_p, feat_p, w1tf, w1tx, w1bx, b1, w2, b2,
                 pw1, pb1, pw2, pb2):
    g = _B * _M
    xyz_g = xyz_p.reshape(g, _PRED_R, 3)
    xyzt_g = jnp.transpose(xyz_g, (0, 2, 1))
    feat_g = feat_p.reshape(g, 1, _C)
    wspec = lambda shp: pl.BlockSpec(shp, lambda i: tuple(0 for _ in shp))
    feat_o, xyz_o = pl.pallas_call(
        _pred_body,
        grid=(g,),
        in_specs=[
            pl.BlockSpec((None, _PRED_R, 3), lambda i: (i, 0, 0)),
            pl.BlockSpec((None, 3, _PRED_R), lambda i: (i, 0, 0)),
            pl.BlockSpec((None, 1, _C), lambda i: (i, 0, 0)),
            wspec((_C, _C)), wspec((3, _C)), wspec((3, _C)),
            wspec((1, _C)), wspec((_C, _C)), wspec((1, _C)),
            wspec((_C, _C)), wspec((1, _C)), wspec((_C, 3)), wspec((1, 3)),
        ],
        out_specs=[
            pl.BlockSpec((None, _PRED_R, _C), lambda i: (i, 0, 0)),
            pl.BlockSpec((None, _PRED_R, 3), lambda i: (i, 0, 0)),
        ],
        out_shape=[
            jax.ShapeDtypeStruct((g, _PRED_R, _C), jnp.float32),
            jax.ShapeDtypeStruct((g, _PRED_R, 3), jnp.float32),
        ],
    )(xyz_g, xyzt_g, feat_g, w1tf, w1tx, w1bx, b1.reshape(1, _C), w2,
      b2.reshape(1, _C), pw1, pb1.reshape(1, _C), pw2, pb2.reshape(1, 3))
    return feat_o.reshape(_B, _M, _PRED_R, _C), xyz_o.reshape(_B, _M, _PRED_R, 3)


def kernel(ctx_xyz, ctx_tokens, pred_tokens, fp_W, fp_b, ec_W1, ec_b1, ec_W2,
           ec_b2, co_W1, co_b1, co_W2, co_b2, po_W1, po_b1, po_W2, po_b2,
           fo_W1, fo_b1, fo_W2, fo_b2):
    nkey = jax.random.key(42)
    noise = jax.random.normal(jax.random.fold_in(nkey, 0),
                              (_B, _P, _UP_R, 3), dtype=jnp.float32)
    noise = noise / (jnp.linalg.norm(noise, axis=-1, keepdims=True) + 1e-6)
    noise = noise * _CTX_RADIUS
    xyz_ctx = (ctx_xyz[:, :, None, :] + noise).reshape(_B, _N_CTX, 3)

    ctx_feat = _feat_proj(ctx_tokens, fp_W, fp_b)
    feat_ctx = jnp.broadcast_to(
        ctx_feat[:, :, None, :], (_B, _P, _UP_R, _C)
    ).reshape(_B, _N_CTX, _C)

    in_dim = _C + 3
    w1tf = ec_W1[:_C]
    w1tx = ec_W1[_C:in_dim]
    w1bf = ec_W1[in_dim:in_dim + _C]
    w1bx = ec_W1[in_dim + _C:]

    feat1, xyz1 = _edgeconv_big(xyz_ctx, feat_ctx, _N_CTX, w1tf, w1tx, w1bf,
                                w1bx, ec_b1, ec_W2, ec_b2, co_W1, co_b1,
                                co_W2, co_b2)

    anchor = jnp.mean(ctx_xyz, axis=1)
    pns = []
    for m in range(_M):
        pn = jax.random.normal(jax.random.fold_in(nkey, 100 + m),
                               (_B, _PRED_R, 3), dtype=jnp.float32)
        pn = pn / (jnp.linalg.norm(pn, axis=-1, keepdims=True) + 1e-6)
        pns.append(pn * _PRED_RADIUS)
    xyz_p = anchor[:, None, None, :] + jnp.stack(pns, axis=1)
    feat_p, xyz_p_new = _pred_blocks(xyz_p, pred_tokens, w1tf, w1tx, w1bx,
                                     ec_b1, ec_W2, ec_b2, po_W1, po_b1,
                                     po_W2, po_b2)

    pad = _N_ALLP - _N_ALL
    sent = (1e4 + 10.0 * jnp.arange(pad, dtype=jnp.float32))
    sent = jnp.broadcast_to(sent[None, :, None], (_B, pad, 3))
    xyz_all = jnp.concatenate(
        [xyz1, xyz_p_new.reshape(_B, _M * _PRED_R, 3), sent], axis=1)
    feat_all = jnp.concatenate(
        [feat1, feat_p.reshape(_B, _M * _PRED_R, _C),
         jnp.zeros((_B, pad, _C), jnp.float32)], axis=1)

    _, xyz2 = _edgeconv_big(xyz_all, feat_all, _N_ALLP, w1tf, w1tx, w1bf,
                            w1bx, ec_b1, ec_W2, ec_b2, fo_W1, fo_b1, fo_W2,
                            fo_b2)
    return xyz2[:, :_N_ALL]
